# baseline (speedup 1.0000x reference)
import functools

import jax
import jax.numpy as jnp
from jax import lax
from jax.experimental import pallas as pl
from jax.experimental.pallas import tpu as pltpu
from jax.experimental.pallas import tpu_sc as plsc

_NCORES = 2
_NSUB = 16
_NW = _NCORES * _NSUB
_STEP = 512
_J = _STEP // 128


def _flush_sizes(rows_per_tile, cap):
  sizes = []
  r = rows_per_tile
  while r > 0:
    s = min(cap, r)
    sizes.append(s)
    r -= s
  return sizes


@functools.lru_cache(maxsize=None)
def _n_acc(N):
  return -(-(N + 1) // 128) * 128


@functools.lru_cache(maxsize=None)
def _sc_agg(C, N, E_pad, slab):
  per_w = E_pad // _NW
  S = per_w // _STEP
  assert S % 4 == 0
  N_acc = _n_acc(N)
  rows_pt = N_acc // _NSUB
  fsizes = _flush_sizes(rows_pt, _STEP)
  mesh = plsc.VectorSubcoreMesh(core_axis_name="core", subcore_axis_name="sub")

  def body(src_hbm, dst_hbm, table_hbm, out_hbm,
           src_v, dst_v, rows_v, acc, gsem, ssem, isem):
    core = lax.axis_index("core")
    sid = lax.axis_index("sub")
    wid = core * _NSUB + sid
    base = sid * rows_pt

    def zero_acc():
      def fz(i, carry):
        rows_v[1, i] = jnp.zeros((16,), jnp.float32)
        return carry
      lax.fori_loop(0, _STEP, fz, 0)
      off = 0
      for s in fsizes:
        pltpu.sync_copy(rows_v.at[1, pl.ds(0, s)], acc.at[pl.ds(base + off, s)])
        off += s

    def fill_dst3():
      nv = jnp.full((16,), N, jnp.int32)
      for j in range(_J):
        for t in range(8):
          dst_v[3, j, pl.ds(t * 16, 16)] = nv

    def fire_idx(i, q):
      ebase = wid * slab + i * _STEP
      rbase = wid * (slab // 128) + i * _J
      pltpu.async_copy(src_hbm.at[pl.ds(ebase, _STEP)], src_v.at[q], isem)
      pltpu.async_copy(dst_hbm.at[pl.ds(rbase, _J)], dst_v.at[q], isem)

    def drain_i():
      pltpu.make_async_copy(src_hbm.at[pl.ds(0, _STEP)], src_v.at[0],
                            isem).wait()
      pltpu.make_async_copy(dst_hbm.at[pl.ds(0, _J)], dst_v.at[0],
                            isem).wait()

    def offset_add(q, c):
      if c > 0:
        offv = jnp.full((16,), c * N, jnp.int32)
        for t in range(_STEP // 16):
          src_v[q, pl.ds(t * 16, 16)] = src_v[q, pl.ds(t * 16, 16)] + offv

    def fire_gathers(b, q):
      for j in range(_J):
        pltpu.async_copy(table_hbm.at[src_v.at[q, pl.ds(j * 128, 128)]],
                         rows_v.at[b, pl.ds(j * 128, 128)], gsem)

    def fire_scatters(b, q):
      for j in range(_J):
        pltpu.async_copy(rows_v.at[b, pl.ds(j * 128, 128)],
                         acc.at[dst_v.at[q, j]], ssem, add=True)

    def drain_g():
      for _ in range(_J):
        pltpu.make_async_copy(table_hbm.at[src_v.at[0, pl.ds(0, 128)]],
                              rows_v.at[0, pl.ds(0, 128)], gsem).wait()

    def drain_s():
      for _ in range(_J):
        pltpu.make_async_copy(rows_v.at[0, pl.ds(0, 128)],
                              acc.at[dst_v.at[0, 0]], ssem).wait()

    zero_acc()
    plsc.subcore_barrier()

    for c in range(C):
      fill_dst3()
      fire_idx(0, 0)
      drain_i()
      offset_add(0, c)
      fire_gathers(0, 0)
      fire_idx(1, 1)
      fire_scatters(1, 3)

      def quad(i4, carry):
        for q in range(4):
          i = 4 * i4 + q
          b = q % 2
          drain_s()
          drain_i()
          offset_add((q + 1) % 4, c)
          drain_g()
          fire_scatters(b, q)
          fire_gathers(b ^ 1, (q + 1) % 4)
          fire_idx(i + 2, (q + 2) % 4)
        return carry
      lax.fori_loop(0, S // 4, quad, 0)
      drain_s()
      drain_g()
      drain_i()
      plsc.subcore_barrier()
      off = 0
      for s in fsizes:
        pltpu.sync_copy(
            acc.at[pl.ds(base + off, s)],
            out_hbm.at[core, c, pl.ds(base + off, s)])
        off += s
      if c < C - 1:
        zero_acc()
        plsc.subcore_barrier()

  return pl.kernel(
      body,
      out_type=jax.ShapeDtypeStruct((_NCORES, C, N_acc, 16), jnp.float32),
      mesh=mesh,
      compiler_params=pltpu.CompilerParams(use_tc_tiling_on_sc=False),
      scratch_types=[
          pltpu.VMEM((4, _STEP), jnp.int32),
          pltpu.VMEM((4, _J, 128), jnp.int32),
          pltpu.VMEM((2, _STEP, 16), jnp.float32),
          pltpu.VMEM_SHARED((N_acc, 16), jnp.float32),
          pltpu.SemaphoreType.DMA,
          pltpu.SemaphoreType.DMA,
          pltpu.SemaphoreType.DMA,
      ])


@functools.lru_cache(maxsize=None)
def _sc_degree(N, E_pad, slab):
  per_w = E_pad // _NW
  steps = per_w // 1024
  N_acc = _n_acc(N)
  rows_pt = N_acc // _NSUB
  fsizes = _flush_sizes(rows_pt, 1024)
  mesh = plsc.VectorSubcoreMesh(core_axis_name="core", subcore_axis_name="sub")

  def body(dst_hbm, out_hbm, dst_v, rows_v, acc, ssem):
    core = lax.axis_index("core")
    sid = lax.axis_index("sub")
    wid = core * _NSUB + sid

    def fz(i, carry):
      rows_v[i] = jnp.zeros((16,), jnp.float32)
      return carry
    lax.fori_loop(0, 1024, fz, 0)

    base = sid * rows_pt
    off = 0
    for s in fsizes:
      pltpu.sync_copy(rows_v.at[pl.ds(0, s)], acc.at[pl.ds(base + off, s)])
      off += s

    def fo(i, carry):
      rows_v[i] = jnp.full((16,), 1.0, jnp.float32)
      return carry
    lax.fori_loop(0, 1024, fo, 0)
    plsc.subcore_barrier()

    def step(i, carry):
      rbase = wid * (slab // 128) + i * 8
      pltpu.sync_copy(dst_hbm.at[pl.ds(rbase, 8)], dst_v)
      sps = []
      for j in range(8):
        sps.append(pltpu.async_copy(
            rows_v.at[pl.ds(j * 128, 128)],
            acc.at[dst_v.at[j]], ssem, add=True))
      for sp in sps:
        sp.wait()
      return carry
    lax.fori_loop(0, steps, step, 0)
    plsc.subcore_barrier()
    off = 0
    for s in fsizes:
      pltpu.sync_copy(acc.at[pl.ds(base + off, s)],
                      out_hbm.at[core, pl.ds(base + off, s)])
      off += s

  return pl.kernel(
      body,
      out_type=jax.ShapeDtypeStruct((_NCORES, N_acc, 16), jnp.float32),
      mesh=mesh,
      compiler_params=pltpu.CompilerParams(use_tc_tiling_on_sc=False),
      scratch_types=[
          pltpu.VMEM((8, 128), jnp.int32),
          pltpu.VMEM((1024, 16), jnp.float32),
          pltpu.VMEM_SHARED((N_acc, 16), jnp.float32),
          pltpu.SemaphoreType.DMA,
      ])


@functools.lru_cache(maxsize=None)
def _tc_layer(C, NP, NPa, RB, relu, chunked_out):
  grid = -(-NP // RB)

  def body(p_ref, d_ref, h_ref, wl_ref, wr_ref, b_ref, o_ref):
    rdeg = 1.0 / jnp.maximum(d_ref[0] + d_ref[1], 1.0)
    aggs = [(p_ref[0, c] + p_ref[1, c]) * rdeg for c in range(C)]
    if chunked_out:
      for co in range(4):
        acc = b_ref[0, pl.ds(co * 128, 128)] * jnp.ones((RB, 1), jnp.float32)
        for ci in range(C):
          acc += jnp.dot(aggs[ci], wl_ref[pl.ds(ci * 128, 128),
                                          pl.ds(co * 128, 128)],
                         preferred_element_type=jnp.float32)
          acc += jnp.dot(h_ref[ci], wr_ref[pl.ds(ci * 128, 128),
                                           pl.ds(co * 128, 128)],
                         preferred_element_type=jnp.float32)
        if relu:
          acc = jnp.maximum(acc, 0.0)
        o_ref[co] = acc
    else:
      acc = b_ref[...] * jnp.ones((RB, 1), jnp.float32)
      for ci in range(C):
        acc += jnp.dot(h_ref[ci], wr_ref[pl.ds(ci * 128, 128)],
                       preferred_element_type=jnp.float32)
        acc += jnp.dot(aggs[ci], wl_ref[pl.ds(ci * 128, 128)],
                       preferred_element_type=jnp.float32)
      if relu:
        acc = jnp.maximum(acc, 0.0)
      o_ref[...] = acc

  if chunked_out:
    out_shape = jax.ShapeDtypeStruct((4, NP, 128), jnp.float32)
    out_spec = pl.BlockSpec((4, RB, 128), lambda i: (0, i, 0))
    wshape = (C * 128, 512)
  else:
    out_shape = jax.ShapeDtypeStruct((NP, 512), jnp.float32)
    out_spec = pl.BlockSpec((RB, 512), lambda i: (i, 0))
    wshape = (C * 128, 512)

  return pl.pallas_call(
      body,
      grid=(grid,),
      in_specs=[
          pl.BlockSpec((_NCORES, C, RB, 128), lambda i: (0, 0, i, 0)),
          pl.BlockSpec((_NCORES, RB, 128), lambda i: (0, i, 0)),
          pl.BlockSpec((C, RB, 128), lambda i: (0, i, 0)),
          pl.BlockSpec(wshape, lambda i: (0, 0)),
          pl.BlockSpec(wshape, lambda i: (0, 0)),
          pl.BlockSpec((1, 512), lambda i: (0, 0)),
      ],
      out_specs=out_spec,
      out_shape=out_shape,
  )


def _blockdiag_chunked(WT):
  C = WT.shape[0] // 16
  eye8 = jnp.eye(8, dtype=WT.dtype)
  T = WT.reshape(C, 16, 4, 16)
  return jnp.einsum("cioj,kK->ckioKj", T, eye8).reshape(C * 128, 512)


def _blockdiag_full(WT):
  C = WT.shape[0] // 16
  eye8 = jnp.eye(8, dtype=WT.dtype)
  T = WT.reshape(C, 16, 64)
  return jnp.einsum("cij,kK->ckiKj", T, eye8).reshape(C * 128, 512)


def _bias_chunked(b):
  return jnp.tile(b.reshape(4, 1, 16), (1, 8, 1)).reshape(1, 512)


def kernel(x, edge_index, Wl1a, bl1a, Wr1a, Wl1b, bl1b, Wr1b,
           Wl2a, bl2a, Wr2a, Wl2b, bl2b, Wr2b):
  N = x.shape[0]
  E = edge_index.shape[1]
  N_acc = _n_acc(N)
  NP = N // 8
  NPa = N_acc // 8
  RB = 256

  src = edge_index[0].astype(jnp.int32)
  dst = edge_index[1].astype(jnp.int32)
  E_pad = -(-E // (_NW * 2048)) * (_NW * 2048)
  per_w = E_pad // _NW
  slab = per_w + 1024
  src_p = jnp.pad(
      jnp.concatenate([src, jnp.zeros((E_pad - E,), jnp.int32)]
                      ).reshape(_NW, per_w),
      ((0, 0), (0, 1024))).reshape(-1)
  dst_p = jnp.pad(
      jnp.concatenate([dst, jnp.full((E_pad - E,), N, jnp.int32)]
                      ).reshape(_NW, per_w),
      ((0, 0), (0, 1024))).reshape(-1, 128)

  degp = _sc_degree(N, E_pad, slab)(dst_p).reshape(2, NPa, 128)

  agg1 = _sc_agg(1, N, E_pad, slab)
  agg4 = _sc_agg(4, N, E_pad, slab)
  tc1 = _tc_layer(1, NP, NPa, RB, True, True)
  tc4 = _tc_layer(4, NP, NPa, RB, True, True)
  tc4f = _tc_layer(4, NP, NPa, RB, False, False)

  p1 = agg1(src_p, dst_p, x).reshape(2, 1, NPa, 128)
  h1 = tc1(p1, degp, x.reshape(1, NP, 128),
           _blockdiag_chunked(Wl1a.T), _blockdiag_chunked(Wr1a.T),
           _bias_chunked(bl1a))

  p2 = agg4(src_p, dst_p, h1.reshape(4 * N, 16)).reshape(2, 4, NPa, 128)
  h2 = tc4(p2, degp, h1,
           _blockdiag_chunked(Wl1b.T), _blockdiag_chunked(Wr1b.T),
           _bias_chunked(bl1b))

  p3 = agg4(src_p, dst_p, h2.reshape(4 * N, 16)).reshape(2, 4, NPa, 128)
  h3 = tc4(p3, degp, h2,
           _blockdiag_chunked(Wl2a.T), _blockdiag_chunked(Wr2a.T),
           _bias_chunked(bl2a))

  p4 = agg4(src_p, dst_p, h3.reshape(4 * N, 16)).reshape(2, 4, NPa, 128)
  out = tc4f(p4, degp, h3,
             _blockdiag_full(Wl2b.T), _blockdiag_full(Wr2b.T),
             jnp.tile(bl2b, 8).reshape(1, 512))
  return out.reshape(N, 64)


# speedup vs baseline: 2.0602x; 2.0602x over previous
import functools

import jax
import jax.numpy as jnp
from jax import lax
from jax.experimental import pallas as pl
from jax.experimental.pallas import tpu as pltpu
from jax.experimental.pallas import tpu_sc as plsc

_NCORES = 2
_NSUB = 16
_NW = _NCORES * _NSUB
_STEP = 512
_J = _STEP // 128


def _flush_sizes(rows_per_tile, cap):
  sizes = []
  r = rows_per_tile
  while r > 0:
    s = min(cap, r)
    sizes.append(s)
    r -= s
  return sizes


@functools.lru_cache(maxsize=None)
def _n_acc(N):
  return -(-(N + 1) // 128) * 128


@functools.lru_cache(maxsize=None)
def _sc_agg(C, N, E_pad):
  per_w = E_pad // _NW
  S = per_w // _STEP
  assert S % 2 == 0
  N_acc = _n_acc(N)
  rows_pt = N_acc // _NSUB
  fsizes = _flush_sizes(rows_pt, _STEP)
  mesh = plsc.VectorSubcoreMesh(core_axis_name="core", subcore_axis_name="sub")

  def body(src_hbm, dst_hbm, table_hbm, out_hbm,
           src_v, dst_v, rows_v, acc, gsem, ssem, isem):
    core = lax.axis_index("core")
    sid = lax.axis_index("sub")
    wid = core * _NSUB + sid
    base = sid * rows_pt

    def zero_acc():
      def fz(i, carry):
        rows_v[1, i] = jnp.zeros((16,), jnp.float32)
        return carry
      lax.fori_loop(0, _STEP, fz, 0)
      off = 0
      for s in fsizes:
        pltpu.sync_copy(rows_v.at[1, pl.ds(0, s)], acc.at[pl.ds(base + off, s)])
        off += s

    def fill_dst1():
      nv = jnp.full((16,), N, jnp.int32)
      for j in range(_J):
        for t in range(8):
          dst_v[1, j, pl.ds(t * 16, 16)] = nv

    def load_idx(i, b, c):
      ebase = wid * per_w + i * _STEP
      rbase = wid * (per_w // 128) + i * _J
      cpd = pltpu.async_copy(dst_hbm.at[pl.ds(rbase, _J)], dst_v.at[b], isem)
      cps = pltpu.async_copy(src_hbm.at[pl.ds(ebase, _STEP)], src_v.at[b], isem)
      cpd.wait()
      cps.wait()
      if c > 0:
        offv = jnp.full((16,), c * N, jnp.int32)
        for t in range(_STEP // 16):
          src_v[b, pl.ds(t * 16, 16)] = src_v[b, pl.ds(t * 16, 16)] + offv

    def fire_gathers(b):
      for j in range(_J):
        pltpu.async_copy(table_hbm.at[src_v.at[b, pl.ds(j * 128, 128)]],
                         rows_v.at[b, pl.ds(j * 128, 128)], gsem)

    def fire_scatters(b):
      for j in range(_J):
        pltpu.async_copy(rows_v.at[b, pl.ds(j * 128, 128)],
                         acc.at[dst_v.at[b, j]], ssem, add=True)

    def drain_g():
      for _ in range(_J):
        pltpu.make_async_copy(table_hbm.at[src_v.at[0, pl.ds(0, 128)]],
                              rows_v.at[0, pl.ds(0, 128)], gsem).wait()

    def drain_s():
      for _ in range(_J):
        pltpu.make_async_copy(rows_v.at[0, pl.ds(0, 128)],
                              acc.at[dst_v.at[0, 0]], ssem).wait()

    zero_acc()
    plsc.subcore_barrier()

    for c in range(C):
      fill_dst1()
      load_idx(0, 0, c)
      fire_gathers(0)
      fire_scatters(1)

      def pair(i2, carry):
        for b in (0, 1):
          i = 2 * i2 + b
          drain_s()
          load_idx(i + 1, b ^ 1, c)
          drain_g()
          fire_scatters(b)
          fire_gathers(b ^ 1)
        return carry
      lax.fori_loop(0, S // 2, pair, 0)
      drain_s()
      drain_g()
      plsc.subcore_barrier()
      off = 0
      for s in fsizes:
        pltpu.sync_copy(
            acc.at[pl.ds(base + off, s)],
            out_hbm.at[core, c, pl.ds(base + off, s)])
        off += s
      if c < C - 1:
        zero_acc()
        plsc.subcore_barrier()

  return pl.kernel(
      body,
      out_type=jax.ShapeDtypeStruct((_NCORES, C, N_acc, 16), jnp.float32),
      mesh=mesh,
      compiler_params=pltpu.CompilerParams(use_tc_tiling_on_sc=False),
      scratch_types=[
          pltpu.VMEM((2, _STEP), jnp.int32),
          pltpu.VMEM((2, _J, 128), jnp.int32),
          pltpu.VMEM((2, _STEP, 16), jnp.float32),
          pltpu.VMEM_SHARED((N_acc, 16), jnp.float32),
          pltpu.SemaphoreType.DMA,
          pltpu.SemaphoreType.DMA,
          pltpu.SemaphoreType.DMA,
      ])


@functools.lru_cache(maxsize=None)
def _sc_degree(N, E_pad):
  per_w = E_pad // _NW
  steps = per_w // 1024
  N_acc = _n_acc(N)
  rows_pt = N_acc // _NSUB
  fsizes = _flush_sizes(rows_pt, 1024)
  mesh = plsc.VectorSubcoreMesh(core_axis_name="core", subcore_axis_name="sub")

  def body(dst_hbm, out_hbm, dst_v, rows_v, acc, ssem):
    core = lax.axis_index("core")
    sid = lax.axis_index("sub")
    wid = core * _NSUB + sid

    def fz(i, carry):
      rows_v[i] = jnp.zeros((16,), jnp.float32)
      return carry
    lax.fori_loop(0, 1024, fz, 0)

    base = sid * rows_pt
    off = 0
    for s in fsizes:
      pltpu.sync_copy(rows_v.at[pl.ds(0, s)], acc.at[pl.ds(base + off, s)])
      off += s

    def fo(i, carry):
      rows_v[i] = jnp.full((16,), 1.0, jnp.float32)
      return carry
    lax.fori_loop(0, 1024, fo, 0)
    plsc.subcore_barrier()

    def step(i, carry):
      rbase = wid * (per_w // 128) + i * 8
      pltpu.sync_copy(dst_hbm.at[pl.ds(rbase, 8)], dst_v)
      sps = []
      for j in range(8):
        sps.append(pltpu.async_copy(
            rows_v.at[pl.ds(j * 128, 128)],
            acc.at[dst_v.at[j]], ssem, add=True))
      for sp in sps:
        sp.wait()
      return carry
    lax.fori_loop(0, steps, step, 0)
    plsc.subcore_barrier()
    off = 0
    for s in fsizes:
      pltpu.sync_copy(acc.at[pl.ds(base + off, s)],
                      out_hbm.at[core, pl.ds(base + off, s)])
      off += s

  return pl.kernel(
      body,
      out_type=jax.ShapeDtypeStruct((_NCORES, N_acc, 16), jnp.float32),
      mesh=mesh,
      compiler_params=pltpu.CompilerParams(use_tc_tiling_on_sc=False),
      scratch_types=[
          pltpu.VMEM((8, 128), jnp.int32),
          pltpu.VMEM((1024, 16), jnp.float32),
          pltpu.VMEM_SHARED((N_acc, 16), jnp.float32),
          pltpu.SemaphoreType.DMA,
      ])


@functools.lru_cache(maxsize=None)
def _tc_layer(C, NP, NPa, RB, relu, chunked_out):
  grid = -(-NP // RB)

  def body(p_ref, d_ref, h_ref, wl_ref, wr_ref, b_ref, o_ref):
    rdeg = 1.0 / jnp.maximum(d_ref[0] + d_ref[1], 1.0)
    aggs = [(p_ref[0, c] + p_ref[1, c]) * rdeg for c in range(C)]
    if chunked_out:
      for co in range(4):
        acc = b_ref[0, pl.ds(co * 128, 128)] * jnp.ones((RB, 1), jnp.float32)
        for ci in range(C):
          acc += jnp.dot(aggs[ci], wl_ref[pl.ds(ci * 128, 128),
                                          pl.ds(co * 128, 128)],
                         preferred_element_type=jnp.float32)
          acc += jnp.dot(h_ref[ci], wr_ref[pl.ds(ci * 128, 128),
                                           pl.ds(co * 128, 128)],
                         preferred_element_type=jnp.float32)
        if relu:
          acc = jnp.maximum(acc, 0.0)
        o_ref[co] = acc
    else:
      acc = b_ref[...] * jnp.ones((RB, 1), jnp.float32)
      for ci in range(C):
        acc += jnp.dot(h_ref[ci], wr_ref[pl.ds(ci * 128, 128)],
                       preferred_element_type=jnp.float32)
        acc += jnp.dot(aggs[ci], wl_ref[pl.ds(ci * 128, 128)],
                       preferred_element_type=jnp.float32)
      if relu:
        acc = jnp.maximum(acc, 0.0)
      o_ref[...] = acc

  if chunked_out:
    out_shape = jax.ShapeDtypeStruct((4, NP, 128), jnp.float32)
    out_spec = pl.BlockSpec((4, RB, 128), lambda i: (0, i, 0))
    wshape = (C * 128, 512)
  else:
    out_shape = jax.ShapeDtypeStruct((NP, 512), jnp.float32)
    out_spec = pl.BlockSpec((RB, 512), lambda i: (i, 0))
    wshape = (C * 128, 512)

  return pl.pallas_call(
      body,
      grid=(grid,),
      in_specs=[
          pl.BlockSpec((_NCORES, C, RB, 128), lambda i: (0, 0, i, 0)),
          pl.BlockSpec((_NCORES, RB, 128), lambda i: (0, i, 0)),
          pl.BlockSpec((C, RB, 128), lambda i: (0, i, 0)),
          pl.BlockSpec(wshape, lambda i: (0, 0)),
          pl.BlockSpec(wshape, lambda i: (0, 0)),
          pl.BlockSpec((1, 512), lambda i: (0, 0)),
      ],
      out_specs=out_spec,
      out_shape=out_shape,
  )


def _blockdiag_chunked(WT):
  C = WT.shape[0] // 16
  eye8 = jnp.eye(8, dtype=WT.dtype)
  T = WT.reshape(C, 16, 4, 16)
  return jnp.einsum("cioj,kK->ckioKj", T, eye8).reshape(C * 128, 512)


def _blockdiag_full(WT):
  C = WT.shape[0] // 16
  eye8 = jnp.eye(8, dtype=WT.dtype)
  T = WT.reshape(C, 16, 64)
  return jnp.einsum("cij,kK->ckiKj", T, eye8).reshape(C * 128, 512)


def _bias_chunked(b):
  return jnp.tile(b.reshape(4, 1, 16), (1, 8, 1)).reshape(1, 512)


def kernel(x, edge_index, Wl1a, bl1a, Wr1a, Wl1b, bl1b, Wr1b,
           Wl2a, bl2a, Wr2a, Wl2b, bl2b, Wr2b):
  N = x.shape[0]
  E = edge_index.shape[1]
  N_acc = _n_acc(N)
  NP = N // 8
  NPa = N_acc // 8
  RB = 256

  src = edge_index[0].astype(jnp.int32)
  dst = edge_index[1].astype(jnp.int32)
  E_pad = -(-E // (_NW * 1024)) * (_NW * 1024)
  src_p = jnp.concatenate([src, jnp.zeros((E_pad - E + 2 * _STEP,), jnp.int32)])
  dst_p = jnp.concatenate(
      [dst, jnp.full((E_pad - E + 2 * _STEP,), N, jnp.int32)]).reshape(-1, 128)

  degp = _sc_degree(N, E_pad)(dst_p).reshape(2, NPa, 128)

  agg1 = _sc_agg(1, N, E_pad)
  agg4 = _sc_agg(4, N, E_pad)
  tc1 = _tc_layer(1, NP, NPa, RB, True, True)
  tc4 = _tc_layer(4, NP, NPa, RB, True, True)
  tc4f = _tc_layer(4, NP, NPa, RB, False, False)

  p1 = agg1(src_p, dst_p, x).reshape(2, 1, NPa, 128)
  h1 = tc1(p1, degp, x.reshape(1, NP, 128),
           _blockdiag_chunked(Wl1a.T), _blockdiag_chunked(Wr1a.T),
           _bias_chunked(bl1a))

  p2 = agg4(src_p, dst_p, h1.reshape(4 * N, 16)).reshape(2, 4, NPa, 128)
  h2 = tc4(p2, degp, h1,
           _blockdiag_chunked(Wl1b.T), _blockdiag_chunked(Wr1b.T),
           _bias_chunked(bl1b))

  p3 = agg4(src_p, dst_p, h2.reshape(4 * N, 16)).reshape(2, 4, NPa, 128)
  h3 = tc4(p3, degp, h2,
           _blockdiag_chunked(Wl2a.T), _blockdiag_chunked(Wr2a.T),
           _bias_chunked(bl2a))

  p4 = agg4(src_p, dst_p, h3.reshape(4 * N, 16)).reshape(2, 4, NPa, 128)
  out = tc4f(p4, degp, h3,
             _blockdiag_full(Wl2b.T), _blockdiag_full(Wr2b.T),
             jnp.tile(bl2b, 8).reshape(1, 512))
  return out.reshape(N, 64)
